# Initial kernel scaffold; baseline (speedup 1.0000x reference)
import jax
import jax.numpy as jnp
from jax.experimental import pallas as pl


def kernel(global_attr, node_attr, edge_attr, edges, node_idx, edge_idx, W, b):
    raise NotImplementedError("write your pallas kernel here")


# speedup vs baseline: 4.8621x; 4.8621x over previous
import functools

import jax
import jax.numpy as jnp
from jax import lax
from jax.experimental import pallas as pl
from jax.experimental.pallas import tpu as pltpu
from jax.experimental.pallas import tpu_sc as plsc

B = 64
N = 50000
E = 800000
D_F = 128
D_E = 16
D_G = 128
D_OUT = 128

NC = 2
NS = 16
NW = NC * NS

NODE_S = 256
EDGE_S = 512
NODE_CHUNKS = -(-N // NODE_S)
EDGE_CHUNKS = -(-E // EDGE_S)
NODE_TAIL = N - (NODE_CHUNKS - 1) * NODE_S
EDGE_TAIL = E - (EDGE_CHUNKS - 1) * EDGE_S
NODE_PAD = NODE_CHUNKS * NODE_S
EDGE_PAD = EDGE_CHUNKS * EDGE_S
ACC_ROWS = B + 1

_mesh = plsc.VectorSubcoreMesh(
    core_axis_name="c", subcore_axis_name="s", num_cores=NC, num_subcores=NS
)


def _zero_rows(ref, nrows, ncol_groups):
    z = jnp.zeros((16,), jnp.float32)

    def body(r, carry):
        for g in range(ncol_groups):
            ref[r, pl.ds(g * 16, 16)] = z
        return carry

    lax.fori_loop(0, nrows, body, 0)


@functools.partial(
    pl.kernel,
    out_type=(
        jax.ShapeDtypeStruct((NW, B, D_F), jnp.float32),
        jax.ShapeDtypeStruct((NW, B, D_E), jnp.float32),
    ),
    mesh=_mesh,
    scratch_types=[
        pltpu.VMEM((NODE_S, D_F), jnp.float32),
        pltpu.VMEM((NODE_S,), jnp.int32),
        pltpu.VMEM((EDGE_S, D_E), jnp.float32),
        pltpu.VMEM((EDGE_S,), jnp.int32),
        pltpu.VMEM_SHARED((NS, ACC_ROWS, D_F), jnp.float32),
        pltpu.VMEM_SHARED((NS, ACC_ROWS, D_E), jnp.float32),
    ],
)
def _sc_segsum(
    node_hbm, nidx_hbm, edge_hbm, eidx_hbm,
    npart_hbm, epart_hbm,
    nrows_v, nidx_v, erows_v, eidx_v, nacc_sh, eacc_sh,
):
    cid = lax.axis_index("c")
    sid = lax.axis_index("s")
    wid = cid * NS + sid

    _zero_rows(nrows_v, ACC_ROWS, D_F // 16)
    _zero_rows(erows_v, ACC_ROWS, D_E // 16)
    pltpu.sync_copy(nrows_v.at[pl.ds(0, ACC_ROWS)], nacc_sh.at[sid])
    pltpu.sync_copy(erows_v.at[pl.ds(0, ACC_ROWS)], eacc_sh.at[sid])

    def seg_loop(attr_hbm, idx_hbm, rows_v, idx_v, acc_sh, chunk, nchunks, tail):
        subrows = chunk // 128

        def body(k, carry):
            c = wid + k * NW

            @pl.when(c < nchunks)
            def _():
                pltpu.sync_copy(idx_hbm.at[pl.ds(c * chunk, chunk)], idx_v)
                if tail == chunk:
                    pltpu.sync_copy(attr_hbm.at[pl.ds(c * chunk, chunk)], rows_v)
                else:
                    @pl.when(c < nchunks - 1)
                    def _():
                        pltpu.sync_copy(attr_hbm.at[pl.ds(c * chunk, chunk)], rows_v)

                    @pl.when(c == nchunks - 1)
                    def _():
                        pltpu.sync_copy(
                            attr_hbm.at[pl.ds(c * chunk, tail)],
                            rows_v.at[pl.ds(0, tail)],
                        )

                for j in range(subrows):
                    pltpu.sync_copy(
                        rows_v.at[pl.ds(j * 128, 128)],
                        acc_sh.at[sid].at[idx_v.at[pl.ds(j * 128, 128)]],
                        add=True,
                    )

            return carry

        lax.fori_loop(0, -(-nchunks // NW), body, 0)

    seg_loop(node_hbm, nidx_hbm, nrows_v, nidx_v, nacc_sh, NODE_S, NODE_CHUNKS, NODE_TAIL)
    seg_loop(edge_hbm, eidx_hbm, erows_v, eidx_v, eacc_sh, EDGE_S, EDGE_CHUNKS, EDGE_TAIL)

    pltpu.sync_copy(nacc_sh.at[sid].at[pl.ds(0, B)], nrows_v.at[pl.ds(0, B)])
    pltpu.sync_copy(nrows_v.at[pl.ds(0, B)], npart_hbm.at[wid])
    pltpu.sync_copy(eacc_sh.at[sid].at[pl.ds(0, B)], erows_v.at[pl.ds(0, B)])
    pltpu.sync_copy(erows_v.at[pl.ds(0, B)], epart_hbm.at[wid])


def _finish_body(g_ref, np_ref, ep_ref, w_ref, b_ref, o_ref):
    nacc = jnp.sum(np_ref[...], axis=0)
    eacc = jnp.sum(ep_ref[...], axis=0)
    out = jnp.dot(g_ref[...], w_ref[pl.ds(0, D_G), :],
                  preferred_element_type=jnp.float32)
    out += jnp.dot(nacc, w_ref[pl.ds(D_G, D_F), :],
                   preferred_element_type=jnp.float32)
    out += jnp.dot(eacc, w_ref[pl.ds(D_G + D_F, D_E), :],
                   preferred_element_type=jnp.float32)
    o_ref[...] = out + b_ref[...]


_finish = pl.pallas_call(
    _finish_body,
    out_shape=jax.ShapeDtypeStruct((B, D_OUT), jnp.float32),
)


def kernel(global_attr, node_attr, edge_attr, edges, node_idx, edge_idx, W, b):
    del edges
    nidx = node_idx.astype(jnp.int32)
    eidx = edge_idx.astype(jnp.int32)
    nidx_p = jnp.concatenate([nidx, jnp.full((NODE_PAD - N,), B, jnp.int32)])
    eidx_p = jnp.concatenate([eidx, jnp.full((EDGE_PAD - E,), B, jnp.int32)])

    npart, epart = _sc_segsum(node_attr, nidx_p, edge_attr, eidx_p)
    return _finish(global_attr, npart, epart, W, b.reshape(1, D_OUT))


# speedup vs baseline: 5.5219x; 1.1357x over previous
import functools

import jax
import jax.numpy as jnp
from jax import lax
from jax.experimental import pallas as pl
from jax.experimental.pallas import tpu as pltpu
from jax.experimental.pallas import tpu_sc as plsc

B = 64
N = 50000
E = 800000
D_F = 128
D_E = 16
D_G = 128
D_OUT = 128

NC = 2
NS = 16
NW = NC * NS

NODE_S = 128
EDGE_S = 256
NODE_CHUNKS = -(-N // NODE_S)
EDGE_CHUNKS = -(-E // EDGE_S)
NODE_TAIL = N - (NODE_CHUNKS - 1) * NODE_S
EDGE_TAIL = E - (EDGE_CHUNKS - 1) * EDGE_S
NODE_PAD = NODE_CHUNKS * NODE_S
EDGE_PAD = EDGE_CHUNKS * EDGE_S
ACC_ROWS = B + 1

_mesh = plsc.VectorSubcoreMesh(
    core_axis_name="c", subcore_axis_name="s", num_cores=NC, num_subcores=NS
)


def _zero_rows(ref, nrows, ncol_groups):
    z = jnp.zeros((16,), jnp.float32)

    def body(r, carry):
        for g in range(ncol_groups):
            ref[r, pl.ds(g * 16, 16)] = z
        return carry

    lax.fori_loop(0, nrows, body, 0)


@functools.partial(
    pl.kernel,
    out_type=(
        jax.ShapeDtypeStruct((NW, B, D_F), jnp.float32),
        jax.ShapeDtypeStruct((NW, B, D_E), jnp.float32),
    ),
    mesh=_mesh,
    scratch_types=[
        pltpu.VMEM((NODE_S, D_F), jnp.float32),
        pltpu.VMEM((NODE_S, D_F), jnp.float32),
        pltpu.VMEM((NODE_S,), jnp.int32),
        pltpu.VMEM((NODE_S,), jnp.int32),
        pltpu.VMEM((EDGE_S, D_E), jnp.float32),
        pltpu.VMEM((EDGE_S, D_E), jnp.float32),
        pltpu.VMEM((EDGE_S,), jnp.int32),
        pltpu.VMEM((EDGE_S,), jnp.int32),
        pltpu.VMEM_SHARED((NS, ACC_ROWS, D_F), jnp.float32),
        pltpu.VMEM_SHARED((NS, ACC_ROWS, D_E), jnp.float32),
        pltpu.SemaphoreType.DMA,
        pltpu.SemaphoreType.DMA,
        pltpu.SemaphoreType.DMA,
        pltpu.SemaphoreType.DMA,
        pltpu.SemaphoreType.DMA,
    ],
)
def _sc_segsum(
    node_hbm, nidx_hbm, edge_hbm, eidx_hbm,
    npart_hbm, epart_hbm,
    nrows0, nrows1, nidx0, nidx1, erows0, erows1, eidx0, eidx1,
    nacc_sh, eacc_sh,
    in_sem0, in_sem1, in_sem2, in_sem3, sc_sem,
):
    cid = lax.axis_index("c")
    sid = lax.axis_index("s")
    wid = cid * NS + sid

    _zero_rows(nrows0, ACC_ROWS, D_F // 16)
    _zero_rows(erows0, ACC_ROWS, D_E // 16)
    pltpu.sync_copy(nrows0.at[pl.ds(0, ACC_ROWS)], nacc_sh.at[sid])
    pltpu.sync_copy(erows0.at[pl.ds(0, ACC_ROWS)], eacc_sh.at[sid])

    def seg_loop(attr_hbm, idx_hbm, rows_b, idx_b, in_sems, acc_sh,
                 chunk, nchunks, tail):
        subrows = chunk // 128
        nk = -(-nchunks // NW)
        nk2 = -(-nk // 2)

        def in_copies(c, rows_v, idx_v, sem):
            yield pltpu.make_async_copy(
                idx_hbm.at[pl.ds(c * chunk, chunk)], idx_v, sem)
            if tail == chunk:
                yield pltpu.make_async_copy(
                    attr_hbm.at[pl.ds(c * chunk, chunk)], rows_v, sem)

        def start_in(c, rows_v, idx_v, sem):
            @pl.when(c < nchunks)
            def _():
                for cp in in_copies(c, rows_v, idx_v, sem):
                    cp.start()
                if tail != chunk:
                    @pl.when(c < nchunks - 1)
                    def _():
                        pltpu.async_copy(
                            attr_hbm.at[pl.ds(c * chunk, chunk)], rows_v, sem)

                    @pl.when(c == nchunks - 1)
                    def _():
                        pltpu.async_copy(
                            attr_hbm.at[pl.ds(c * chunk, tail)],
                            rows_v.at[pl.ds(0, tail)], sem)

        def process(c, rows_v, idx_v, sem):
            @pl.when(c < nchunks)
            def _():
                for cp in in_copies(c, rows_v, idx_v, sem):
                    cp.wait()
                if tail != chunk:
                    @pl.when(c < nchunks - 1)
                    def _():
                        pltpu.make_async_copy(
                            attr_hbm.at[pl.ds(c * chunk, chunk)], rows_v, sem
                        ).wait()

                    @pl.when(c == nchunks - 1)
                    def _():
                        pltpu.make_async_copy(
                            attr_hbm.at[pl.ds(c * chunk, tail)],
                            rows_v.at[pl.ds(0, tail)], sem,
                        ).wait()

                descs = []
                for j in range(subrows):
                    descs.append(pltpu.async_copy(
                        rows_v.at[pl.ds(j * 128, 128)],
                        acc_sh.at[sid].at[idx_v.at[pl.ds(j * 128, 128)]],
                        sc_sem, add=True))
                for d in descs:
                    d.wait()

            start_in(c + 2 * NW, rows_v, idx_v, sem)

        start_in(wid, rows_b[0], idx_b[0], in_sems[0])
        start_in(wid + NW, rows_b[1], idx_b[1], in_sems[1])

        def body(k2, carry):
            c0 = wid + (2 * k2) * NW
            process(c0, rows_b[0], idx_b[0], in_sems[0])
            process(c0 + NW, rows_b[1], idx_b[1], in_sems[1])
            return carry

        lax.fori_loop(0, nk2, body, 0)

    seg_loop(node_hbm, nidx_hbm, (nrows0, nrows1), (nidx0, nidx1),
             (in_sem0, in_sem1), nacc_sh, NODE_S, NODE_CHUNKS, NODE_TAIL)
    seg_loop(edge_hbm, eidx_hbm, (erows0, erows1), (eidx0, eidx1),
             (in_sem2, in_sem3), eacc_sh, EDGE_S, EDGE_CHUNKS, EDGE_TAIL)

    pltpu.sync_copy(nacc_sh.at[sid].at[pl.ds(0, B)], nrows0.at[pl.ds(0, B)])
    pltpu.sync_copy(nrows0.at[pl.ds(0, B)], npart_hbm.at[wid])
    pltpu.sync_copy(eacc_sh.at[sid].at[pl.ds(0, B)], erows0.at[pl.ds(0, B)])
    pltpu.sync_copy(erows0.at[pl.ds(0, B)], epart_hbm.at[wid])


def _finish_body(g_ref, np_ref, ep_ref, w_ref, b_ref, o_ref):
    nacc = jnp.sum(np_ref[...], axis=0)
    eacc = jnp.sum(ep_ref[...], axis=0)
    out = jnp.dot(g_ref[...], w_ref[pl.ds(0, D_G), :],
                  preferred_element_type=jnp.float32)
    out += jnp.dot(nacc, w_ref[pl.ds(D_G, D_F), :],
                   preferred_element_type=jnp.float32)
    out += jnp.dot(eacc, w_ref[pl.ds(D_G + D_F, D_E), :],
                   preferred_element_type=jnp.float32)
    o_ref[...] = out + b_ref[...]


_finish = pl.pallas_call(
    _finish_body,
    out_shape=jax.ShapeDtypeStruct((B, D_OUT), jnp.float32),
)


def kernel(global_attr, node_attr, edge_attr, edges, node_idx, edge_idx, W, b):
    del edges
    nidx = node_idx.astype(jnp.int32)
    eidx = edge_idx.astype(jnp.int32)
    nidx_p = jnp.concatenate([nidx, jnp.full((NODE_PAD - N,), B, jnp.int32)])
    eidx_p = jnp.concatenate([eidx, jnp.full((EDGE_PAD - E,), B, jnp.int32)])

    npart, epart = _sc_segsum(node_attr, nidx_p, edge_attr, eidx_p)
    return _finish(global_attr, npart, epart, W, b.reshape(1, D_OUT))
